# baseline (speedup 1.0000x reference)
import functools

import jax
import jax.numpy as jnp
from jax import lax
from jax.experimental import pallas as pl
from jax.experimental.pallas import tpu as pltpu
from jax.experimental.pallas import tpu_sc as plsc

_B, _C, _H, _W = 256, 3, 224, 224
_E, _N = 16, 1000
_HW = _H * _W
_BLK_H = 8
_NSTEPS = _H // _BLK_H
_L = 16
_NP = 1024
_NCHUNK = _NP // _L
_NW = 32
_SPW = _B // _NW


def _gap_route_body(xt_ref, Wg_ref, bg_ref, pt_ref, w_ref, ei_ref, acc_ref):
    i = pl.program_id(0)
    s = jnp.sum(xt_ref[...], axis=(1, 2))

    @pl.when(i == 0)
    def _init():
        acc_ref[...] = s

    @pl.when(i > 0)
    def _accum():
        acc_ref[...] += s

    @pl.when(i == _NSTEPS - 1)
    def _finalize():
        pooled_t = acc_ref[...] * (1.0 / _HW)
        pt_ref[...] = pooled_t
        logits = lax.dot_general(pooled_t, Wg_ref[...],
                                 (((0,), (0,)), ((), ())),
                                 preferred_element_type=jnp.float32)
        logits = logits + bg_ref[...]
        weights = jax.nn.softmax(logits, axis=1)
        w_ref[...] = weights
        m = jnp.max(weights, axis=1, keepdims=True)
        lane = lax.broadcasted_iota(jnp.int32, (_B, _E), 1)
        ei_ref[...] = jnp.min(jnp.where(weights == m, lane, _E), axis=1,
                              keepdims=True)


def _gap_route(xt, Wg, bg2):
    return pl.pallas_call(
        _gap_route_body,
        grid=(_NSTEPS,),
        in_specs=[
            pl.BlockSpec((_C, _BLK_H, _W, _B), lambda i: (0, i, 0, 0)),
            pl.BlockSpec((_C, _E), lambda i: (0, 0)),
            pl.BlockSpec((1, _E), lambda i: (0, 0)),
        ],
        out_specs=[
            pl.BlockSpec((_C, _B), lambda i: (0, 0)),
            pl.BlockSpec((_B, _E), lambda i: (0, 0)),
            pl.BlockSpec((_B, 1), lambda i: (0, 0)),
        ],
        out_shape=[
            jax.ShapeDtypeStruct((_C, _B), jnp.float32),
            jax.ShapeDtypeStruct((_B, _E), jnp.float32),
            jax.ShapeDtypeStruct((_B, 1), jnp.int32),
        ],
        scratch_shapes=[pltpu.VMEM((_C, _B), jnp.float32)],
    )(xt, Wg, bg2)


def _combine_body(pooled_hbm, eidx_hbm, wep_hbm, bep_hbm, out_hbm,
                  pooled_v, idx_v, rows_v, bes_v, out_v, sem0, sem1):
    wid = lax.axis_index("s") * 2 + lax.axis_index("c")
    base = wid * _SPW
    pltpu.sync_copy(pooled_hbm, pooled_v)
    pltpu.sync_copy(eidx_hbm.at[pl.ds(base, _SPW)], idx_v)
    cp_rows = pltpu.async_copy(wep_hbm.at[idx_v], rows_v, sem0)
    cp_bes = pltpu.async_copy(bep_hbm.at[idx_v], bes_v, sem1)
    ps = []
    for j in range(_SPW):
        bidx = jnp.full((_L,), base + j, jnp.int32)
        ps.append([plsc.load_gather(pooled_v,
                                    [jnp.full((_L,), c, jnp.int32), bidx])
                   for c in range(_C)])
    cp_rows.wait()
    cp_bes.wait()
    for j in range(_SPW):
        p0, p1, p2 = ps[j]
        for n in range(_NCHUNK):
            t0 = p0 * rows_v[j, pl.ds(n * _L, _L)]
            t1 = p1 * rows_v[j, pl.ds(_NP + n * _L, _L)]
            t2 = p2 * rows_v[j, pl.ds(2 * _NP + n * _L, _L)]
            o = (t0 + t1) + (t2 + bes_v[j, pl.ds(n * _L, _L)])
            out_v[j, pl.ds(n * _L, _L)] = o
    pltpu.sync_copy(out_v, out_hbm.at[pl.ds(base, _SPW)])


def kernel(x, Wg, bg, We, be):
    xt = jnp.transpose(x, (1, 2, 3, 0))
    bg2 = bg.reshape(1, _E)
    pooled_t, weights, eidx = _gap_route(xt, Wg, bg2)
    wep = jnp.pad(We, ((0, 0), (0, 0), (0, _NP - _N))).reshape(_E, _C * _NP)
    bep = jnp.pad(be, ((0, 0), (0, _NP - _N)))
    mesh = plsc.VectorSubcoreMesh(core_axis_name="c", subcore_axis_name="s")
    sc = pl.kernel(
        _combine_body,
        out_type=jax.ShapeDtypeStruct((_B, _NP), jnp.float32),
        mesh=mesh,
        scratch_types=[
            pltpu.VMEM((_C, _B), jnp.float32),
            pltpu.VMEM((_SPW,), jnp.int32),
            pltpu.VMEM((_SPW, _C * _NP), jnp.float32),
            pltpu.VMEM((_SPW, _NP), jnp.float32),
            pltpu.VMEM((_SPW, _NP), jnp.float32),
            pltpu.SemaphoreType.DMA,
            pltpu.SemaphoreType.DMA,
        ],
        compiler_params=pltpu.CompilerParams(needs_layout_passes=False),
    )
    outp = sc(pooled_t, eidx.reshape(_B), wep, bep)
    return (outp[:, :_N], weights)


# speedup vs baseline: 1.0622x; 1.0622x over previous
import functools

import jax
import jax.numpy as jnp
from jax import lax
from jax.experimental import pallas as pl
from jax.experimental.pallas import tpu as pltpu
from jax.experimental.pallas import tpu_sc as plsc

_B, _C, _H, _W = 256, 3, 224, 224
_E, _N = 16, 1000
_HW = _H * _W
_BLK_H = 8
_NSTEPS = _H // _BLK_H
_L = 16
_NP = 1024
_NCHUNK = _NP // _L
_NW = 32
_SPW = _B // _NW


def _gap_route_body(xt_ref, Wg_ref, bg_ref, pt_ref, w_ref, ei_ref, acc_ref):
    i = pl.program_id(0)
    s = jnp.sum(xt_ref[...], axis=(1, 2))

    @pl.when(i == 0)
    def _init():
        acc_ref[...] = s

    @pl.when(i > 0)
    def _accum():
        acc_ref[...] += s

    @pl.when(i == _NSTEPS - 1)
    def _finalize():
        pooled_t = acc_ref[...] * (1.0 / _HW)
        pt_ref[...] = pooled_t
        logits = lax.dot_general(pooled_t, Wg_ref[...],
                                 (((0,), (0,)), ((), ())),
                                 preferred_element_type=jnp.float32)
        logits = logits + bg_ref[...]
        weights = jax.nn.softmax(logits, axis=1)
        w_ref[...] = weights
        m = jnp.max(weights, axis=1, keepdims=True)
        lane = lax.broadcasted_iota(jnp.int32, (_B, _E), 1)
        ei_ref[...] = jnp.min(jnp.where(weights == m, lane, _E), axis=1,
                              keepdims=True)


def _gap_route(xt, Wg, bg2):
    return pl.pallas_call(
        _gap_route_body,
        grid=(_NSTEPS,),
        in_specs=[
            pl.BlockSpec((_C, _BLK_H, _W, _B), lambda i: (0, i, 0, 0)),
            pl.BlockSpec((_C, _E), lambda i: (0, 0)),
            pl.BlockSpec((1, _E), lambda i: (0, 0)),
        ],
        out_specs=[
            pl.BlockSpec((_C, _B), lambda i: (0, 0)),
            pl.BlockSpec((_B, _E), lambda i: (0, 0)),
            pl.BlockSpec((_B, 1), lambda i: (0, 0)),
        ],
        out_shape=[
            jax.ShapeDtypeStruct((_C, _B), jnp.float32),
            jax.ShapeDtypeStruct((_B, _E), jnp.float32),
            jax.ShapeDtypeStruct((_B, 1), jnp.int32),
        ],
        scratch_shapes=[pltpu.VMEM((_C, _B), jnp.float32)],
    )(xt, Wg, bg2)


def _combine_body(pooled_hbm, eidx_hbm, wep_hbm, bep_hbm, out_hbm,
                  pooled_v, idx_v, rows_v, bes_v, out_v, sem0, sem1):
    wid = lax.axis_index("s") * 2 + lax.axis_index("c")
    base = wid * _SPW
    pltpu.sync_copy(pooled_hbm, pooled_v)
    pltpu.sync_copy(eidx_hbm.at[pl.ds(base, _SPW)], idx_v)
    cp_rows = pltpu.async_copy(wep_hbm.at[idx_v], rows_v, sem0)
    cp_bes = pltpu.async_copy(bep_hbm.at[idx_v], bes_v, sem1)
    cp_rows.wait()
    cp_bes.wait()
    for j in range(_SPW):
        bidx = jnp.full((_L,), base + j, jnp.int32)
        p0, p1, p2 = [plsc.load_gather(pooled_v,
                                       [jnp.full((_L,), c, jnp.int32), bidx])
                      for c in range(_C)]

        def _chunk(n, carry, j=j, p0=p0, p1=p1, p2=p2):
            off = n * _L
            t0 = p0 * rows_v[j, pl.ds(off, _L)]
            t1 = p1 * rows_v[j, pl.ds(_NP + off, _L)]
            t2 = p2 * rows_v[j, pl.ds(2 * _NP + off, _L)]
            out_v[j, pl.ds(off, _L)] = (t0 + t1) + (t2 + bes_v[j, pl.ds(off, _L)])
            return carry

        lax.fori_loop(0, _NCHUNK, _chunk, 0, unroll=4)
    pltpu.sync_copy(out_v, out_hbm.at[pl.ds(base, _SPW)])


def kernel(x, Wg, bg, We, be):
    xt = jnp.transpose(x, (1, 2, 3, 0))
    bg2 = bg.reshape(1, _E)
    pooled_t, weights, eidx = _gap_route(xt, Wg, bg2)
    wep = jnp.pad(We, ((0, 0), (0, 0), (0, _NP - _N))).reshape(_E, _C * _NP)
    bep = jnp.pad(be, ((0, 0), (0, _NP - _N)))
    mesh = plsc.VectorSubcoreMesh(core_axis_name="c", subcore_axis_name="s")
    sc = pl.kernel(
        _combine_body,
        out_type=jax.ShapeDtypeStruct((_B, _NP), jnp.float32),
        mesh=mesh,
        scratch_types=[
            pltpu.VMEM((_C, _B), jnp.float32),
            pltpu.VMEM((_SPW,), jnp.int32),
            pltpu.VMEM((_SPW, _C * _NP), jnp.float32),
            pltpu.VMEM((_SPW, _NP), jnp.float32),
            pltpu.VMEM((_SPW, _NP), jnp.float32),
            pltpu.SemaphoreType.DMA,
            pltpu.SemaphoreType.DMA,
        ],
        compiler_params=pltpu.CompilerParams(needs_layout_passes=False),
    )
    outp = sc(pooled_t, eidx.reshape(_B), wep, bep)
    return (outp[:, :_N], weights)


# speedup vs baseline: 1.1135x; 1.0483x over previous
import functools

import jax
import jax.numpy as jnp
from jax import lax
from jax.experimental import pallas as pl
from jax.experimental.pallas import tpu as pltpu
from jax.experimental.pallas import tpu_sc as plsc

_B, _C, _H, _W = 256, 3, 224, 224
_E, _N = 16, 1000
_HW = _H * _W
_BLK_H = 8
_NSTEPS = _H // _BLK_H
_L = 16
_NP = 1024
_NCHUNK = _NP // _L
_NW = 32
_SPW = _B // _NW


def _gap_route_body(xt_ref, Wg_ref, bg_ref, pt_ref, w_ref, ei_ref, acc_ref):
    i = pl.program_id(0)
    s = jnp.sum(xt_ref[...], axis=(1, 2))

    @pl.when(i == 0)
    def _init():
        acc_ref[...] = s

    @pl.when(i > 0)
    def _accum():
        acc_ref[...] += s

    @pl.when(i == _NSTEPS - 1)
    def _finalize():
        pooled_t = acc_ref[...] * (1.0 / _HW)
        pt_ref[...] = pooled_t
        logits = lax.dot_general(pooled_t, Wg_ref[...],
                                 (((0,), (0,)), ((), ())),
                                 preferred_element_type=jnp.float32)
        logits = logits + bg_ref[...]
        weights = jax.nn.softmax(logits, axis=1)
        w_ref[...] = weights
        m = jnp.max(weights, axis=1, keepdims=True)
        lane = lax.broadcasted_iota(jnp.int32, (_B, _E), 1)
        ei_ref[...] = jnp.min(jnp.where(weights == m, lane, _E), axis=1)


def _gap_route(xt, Wg, bg2):
    return pl.pallas_call(
        _gap_route_body,
        grid=(_NSTEPS,),
        in_specs=[
            pl.BlockSpec((_C, _BLK_H, _W, _B), lambda i: (0, i, 0, 0)),
            pl.BlockSpec((_C, _E), lambda i: (0, 0)),
            pl.BlockSpec((1, _E), lambda i: (0, 0)),
        ],
        out_specs=[
            pl.BlockSpec((_C, _B), lambda i: (0, 0)),
            pl.BlockSpec((_B, _E), lambda i: (0, 0)),
            pl.BlockSpec((_B,), lambda i: (0,)),
        ],
        out_shape=[
            jax.ShapeDtypeStruct((_C, _B), jnp.float32),
            jax.ShapeDtypeStruct((_B, _E), jnp.float32),
            jax.ShapeDtypeStruct((_B,), jnp.int32),
        ],
        scratch_shapes=[pltpu.VMEM((_C, _B), jnp.float32)],
    )(xt, Wg, bg2)


def _combine_body(pooled_hbm, eidx_hbm, wep_hbm, bep_hbm, out_hbm,
                  pooled_v, idx_v, rows_v, bes_v, out_v, sem0, sem1):
    wid = lax.axis_index("s") * 2 + lax.axis_index("c")
    base = wid * _SPW
    pltpu.sync_copy(pooled_hbm, pooled_v)
    pltpu.sync_copy(eidx_hbm.at[pl.ds(base, _SPW)], idx_v)
    cp_rows = pltpu.async_copy(wep_hbm.at[idx_v], rows_v, sem0)
    cp_bes = pltpu.async_copy(bep_hbm.at[idx_v], bes_v, sem1)
    cp_rows.wait()
    cp_bes.wait()
    for j in range(_SPW):
        bidx = jnp.full((_L,), base + j, jnp.int32)
        p0, p1, p2 = [plsc.load_gather(pooled_v,
                                       [jnp.full((_L,), c, jnp.int32), bidx])
                      for c in range(_C)]

        def _chunk(n, carry, j=j, p0=p0, p1=p1, p2=p2):
            off = n * _L
            t0 = p0 * rows_v[j, pl.ds(off, _L)]
            t1 = p1 * rows_v[j, pl.ds(_NP + off, _L)]
            t2 = p2 * rows_v[j, pl.ds(2 * _NP + off, _L)]
            out_v[j, pl.ds(off, _L)] = (t0 + t1) + (t2 + bes_v[j, pl.ds(off, _L)])
            return carry

        lax.fori_loop(0, _NCHUNK, _chunk, 0, unroll=8)
    pltpu.sync_copy(out_v, out_hbm.at[pl.ds(base, _SPW)])


def kernel(x, Wg, bg, We, be):
    xt = jnp.transpose(x, (1, 2, 3, 0))
    bg2 = bg.reshape(1, _E)
    pooled_t, weights, eidx = _gap_route(xt, Wg, bg2)
    wep = jnp.pad(We, ((0, 0), (0, 0), (0, _NP - _N))).reshape(_E, _C * _NP)
    bep = jnp.pad(be, ((0, 0), (0, _NP - _N)))
    mesh = plsc.VectorSubcoreMesh(core_axis_name="c", subcore_axis_name="s")
    sc = pl.kernel(
        _combine_body,
        out_type=jax.ShapeDtypeStruct((_B, _NP), jnp.float32),
        mesh=mesh,
        scratch_types=[
            pltpu.VMEM((_C, _B), jnp.float32),
            pltpu.VMEM((_SPW,), jnp.int32),
            pltpu.VMEM((_SPW, _C * _NP), jnp.float32),
            pltpu.VMEM((_SPW, _NP), jnp.float32),
            pltpu.VMEM((_SPW, _NP), jnp.float32),
            pltpu.SemaphoreType.DMA,
            pltpu.SemaphoreType.DMA,
        ],
        compiler_params=pltpu.CompilerParams(needs_layout_passes=False),
    )
    outp = sc(pooled_t, eidx, wep, bep)
    return (outp[:, :_N], weights)


# speedup vs baseline: 1.1403x; 1.0241x over previous
import functools

import jax
import jax.numpy as jnp
from jax import lax
from jax.experimental import pallas as pl
from jax.experimental.pallas import tpu as pltpu
from jax.experimental.pallas import tpu_sc as plsc

_B, _C, _H, _W = 256, 3, 224, 224
_E, _N = 16, 1000
_HW = _H * _W
_BLK_H = 8
_NSTEPS = _H // _BLK_H
_L = 16
_NP = 1024
_NCHUNK = _NP // _L
_NW = 32
_BSC = _B // 2
_SPW = _BSC // _NW


def _gap_route_body(xt_ref, Wg_ref, bg_ref, pt_ref, w_ref, ei_ref, ei2_ref,
                    acc_ref):
    i = pl.program_id(0)
    s = jnp.sum(xt_ref[...], axis=(1, 2))

    @pl.when(i == 0)
    def _init():
        acc_ref[...] = s

    @pl.when(i > 0)
    def _accum():
        acc_ref[...] += s

    @pl.when(i == _NSTEPS - 1)
    def _finalize():
        pooled_t = acc_ref[...] * (1.0 / _HW)
        pt_ref[...] = pooled_t
        logits = lax.dot_general(pooled_t, Wg_ref[...],
                                 (((0,), (0,)), ((), ())),
                                 preferred_element_type=jnp.float32)
        logits = logits + bg_ref[...]
        weights = jax.nn.softmax(logits, axis=1)
        w_ref[...] = weights
        m = jnp.max(weights, axis=1, keepdims=True)
        lane = lax.broadcasted_iota(jnp.int32, (_B, _E), 1)
        ei = jnp.min(jnp.where(weights == m, lane, _E), axis=1, keepdims=True)
        ei_ref[...] = ei[:, 0]
        ei2_ref[...] = ei


def _gap_route(xt, Wg, bg2):
    return pl.pallas_call(
        _gap_route_body,
        grid=(_NSTEPS,),
        in_specs=[
            pl.BlockSpec((_C, _BLK_H, _W, _B), lambda i: (0, i, 0, 0)),
            pl.BlockSpec((_C, _E), lambda i: (0, 0)),
            pl.BlockSpec((1, _E), lambda i: (0, 0)),
        ],
        out_specs=[
            pl.BlockSpec((_C, _B), lambda i: (0, 0)),
            pl.BlockSpec((_B, _E), lambda i: (0, 0)),
            pl.BlockSpec((_B,), lambda i: (0,)),
            pl.BlockSpec((_B, 1), lambda i: (0, 0)),
        ],
        out_shape=[
            jax.ShapeDtypeStruct((_C, _B), jnp.float32),
            jax.ShapeDtypeStruct((_B, _E), jnp.float32),
            jax.ShapeDtypeStruct((_B,), jnp.int32),
            jax.ShapeDtypeStruct((_B, 1), jnp.int32),
        ],
        scratch_shapes=[pltpu.VMEM((_C, _B), jnp.float32)],
    )(xt, Wg, bg2)


def _combine_body(pooled_hbm, eidx_hbm, wep_hbm, bep_hbm, out_hbm,
                  pooled_v, idx8_v, idx_v, rows_v, bes_v, out_v, sem0, sem1):
    wid = lax.axis_index("s") * 2 + lax.axis_index("c")
    base = wid * _SPW
    pltpu.sync_copy(pooled_hbm, pooled_v)
    base8 = (wid // 2) * 8
    pltpu.sync_copy(eidx_hbm.at[pl.ds(base8, 8)], idx8_v)
    lane = lax.iota(jnp.int32, _L)
    off = (wid % 2) * _SPW
    g = plsc.load_gather(idx8_v,
                         [jnp.where(lane < _SPW, lane + off, 0)])
    plsc.store_scatter(idx_v, [lane], g, mask=lane < _SPW)
    cp_rows = pltpu.async_copy(wep_hbm.at[idx_v], rows_v, sem0)
    cp_bes = pltpu.async_copy(bep_hbm.at[idx_v], bes_v, sem1)
    cp_rows.wait()
    cp_bes.wait()
    for j in range(_SPW):
        bidx = jnp.full((_L,), base + j, jnp.int32)
        p0, p1, p2 = [plsc.load_gather(pooled_v,
                                       [jnp.full((_L,), c, jnp.int32), bidx])
                      for c in range(_C)]

        def _chunk(n, carry, j=j, p0=p0, p1=p1, p2=p2):
            off = n * _L
            t0 = p0 * rows_v[j, pl.ds(off, _L)]
            t1 = p1 * rows_v[j, pl.ds(_NP + off, _L)]
            t2 = p2 * rows_v[j, pl.ds(2 * _NP + off, _L)]
            out_v[j, pl.ds(off, _L)] = (t0 + t1) + (t2 + bes_v[j, pl.ds(off, _L)])
            return carry

        lax.fori_loop(0, _NCHUNK, _chunk, 0, unroll=8)
    pltpu.sync_copy(out_v, out_hbm.at[pl.ds(base, _SPW)])


def _tc_combine_body(pt_ref, ei2_ref, We_t_ref, be_ref, out_ref):
    pooled_t = pt_ref[...]
    ec = lax.broadcasted_iota(jnp.int32, (_C, _C), 0)
    eye = (ec == ec.T).astype(jnp.float32)
    pooled = lax.dot_general(pooled_t, eye, (((0,), (0,)), ((), ())),
                             preferred_element_type=jnp.float32)
    lane = lax.broadcasted_iota(jnp.int32, (_B - _BSC, _E), 1)
    onehot = (lane == ei2_ref[...]).astype(jnp.float32)
    acc = jnp.dot(onehot, be_ref[...], preferred_element_type=jnp.float32)
    for c in range(_C):
        mp = onehot * pooled[_BSC:, c:c + 1]
        acc = acc + jnp.dot(mp, We_t_ref[c],
                            preferred_element_type=jnp.float32)
    out_ref[...] = acc


def _tc_combine(pooled_t, eidx2, We_t, be):
    return pl.pallas_call(
        _tc_combine_body,
        grid=(1,),
        in_specs=[
            pl.BlockSpec((_C, _B), lambda i: (0, 0)),
            pl.BlockSpec((_B - _BSC, 1), lambda i: (1, 0)),
            pl.BlockSpec((_C, _E, _N), lambda i: (0, 0, 0)),
            pl.BlockSpec((_E, _N), lambda i: (0, 0)),
        ],
        out_specs=pl.BlockSpec((_B - _BSC, _N), lambda i: (0, 0)),
        out_shape=jax.ShapeDtypeStruct((_B - _BSC, _N), jnp.float32),
    )(pooled_t, eidx2, We_t, be)


def kernel(x, Wg, bg, We, be):
    xt = jnp.transpose(x, (1, 2, 3, 0))
    bg2 = bg.reshape(1, _E)
    pooled_t, weights, eidx, eidx2 = _gap_route(xt, Wg, bg2)
    wep = jnp.pad(We, ((0, 0), (0, 0), (0, _NP - _N))).reshape(_E, _C * _NP)
    bep = jnp.pad(be, ((0, 0), (0, _NP - _N)))
    mesh = plsc.VectorSubcoreMesh(core_axis_name="c", subcore_axis_name="s")
    sc = pl.kernel(
        _combine_body,
        out_type=jax.ShapeDtypeStruct((_BSC, _NP), jnp.float32),
        mesh=mesh,
        scratch_types=[
            pltpu.VMEM((_C, _B), jnp.float32),
            pltpu.VMEM((8,), jnp.int32),
            pltpu.VMEM((_SPW,), jnp.int32),
            pltpu.VMEM((_SPW, _C * _NP), jnp.float32),
            pltpu.VMEM((_SPW, _NP), jnp.float32),
            pltpu.VMEM((_SPW, _NP), jnp.float32),
            pltpu.SemaphoreType.DMA,
            pltpu.SemaphoreType.DMA,
        ],
        compiler_params=pltpu.CompilerParams(needs_layout_passes=False),
    )
    out_a = sc(pooled_t, eidx, wep, bep)
    We_t = We.transpose(1, 0, 2)
    out_b = _tc_combine(pooled_t, eidx2, We_t, be)
    out = jnp.concatenate([out_a[:, :_N], out_b], axis=0)
    return (out, weights)
